# baseline (speedup 1.0000x reference)
import functools

import jax
import jax.numpy as jnp
from jax import lax
from jax.experimental import pallas as pl
from jax.experimental.pallas import tpu as pltpu
from jax.experimental.pallas import tpu_sc as plsc

_N_TOKENS = 32768
_HIDDEN = 4096
_NUM_EXPERTS = 64
_TOP_K = 8
_BR = 1024

_NW = 32
_ROWS_PER_W = _N_TOKENS // _NW
_LOG_W = _ROWS_PER_W * _NUM_EXPERTS
_OUT_W = _ROWS_PER_W * _TOP_K


def _matmul_kernel(x_ref, w_ref, logits_ref):
    logits_ref[...] = jax.lax.dot_general(
        x_ref[...], w_ref[...], (((1,), (1,)), ((), ())),
        preferred_element_type=jnp.float32,
    )


def _tc_logits(x, W):
    return pl.pallas_call(
        _matmul_kernel,
        grid=(_N_TOKENS // _BR,),
        in_specs=[
            pl.BlockSpec((_BR, _HIDDEN), lambda i: (i, 0)),
            pl.BlockSpec((_NUM_EXPERTS, _HIDDEN), lambda i: (0, 0)),
        ],
        out_specs=pl.BlockSpec((_BR, _NUM_EXPERTS), lambda i: (i, 0)),
        out_shape=jax.ShapeDtypeStruct((_N_TOKENS, _NUM_EXPERTS), jnp.float32),
        compiler_params=pltpu.CompilerParams(
            dimension_semantics=("parallel",),
        ),
    )(x, W)


def _merge16(ak, av, bk, bv):
    rbk = lax.rev(bk, (0,))
    rbv = lax.rev(bv, (0,))
    m = ak >= rbk
    hk = jnp.where(m, ak, rbk)
    hv = jnp.where(m, av, rbv)
    return plsc.sort_key_val(hk, hv, descending=True)


def _sc_kernel(logits_hbm, wts_hbm, idx_hbm, log_v, wts_v, idx_v):
    wid = lax.axis_index("s") * 2 + lax.axis_index("c")
    pltpu.sync_copy(logits_hbm.at[pl.ds(wid * _LOG_W, _LOG_W)], log_v)

    lane = lax.broadcasted_iota(jnp.int32, (16,), 0)
    msk8 = lane < _TOP_K

    def row_body(r, carry):
        off = r * _NUM_EXPERTS
        ks, vs = [], []
        for j in range(4):
            v = log_v[pl.ds(off + 16 * j, 16)]
            b = lax.bitcast_convert_type(v, jnp.int32)
            mono = b ^ ((b >> 31) & jnp.int32(0x7FFFFFFF))
            key = (mono & jnp.int32(~63)) | (jnp.int32(63 - 16 * j) - lane)
            sk, sv = plsc.sort_key_val(key, v, descending=True)
            ks.append(sk)
            vs.append(sv)
        k01, v01 = _merge16(ks[0], vs[0], ks[1], vs[1])
        k23, v23 = _merge16(ks[2], vs[2], ks[3], vs[3])
        kf, vf = _merge16(k01, v01, k23, v23)

        idx = jnp.int32(63) - (kf & jnp.int32(63))
        mx = jnp.max(jnp.where(msk8, vf, -jnp.inf))
        e = jnp.exp(jnp.where(msk8, vf - mx, -jnp.inf))
        w = e / jnp.sum(e)

        plsc.store_compressed(wts_v.at[pl.ds(r * _TOP_K, 16)], w, mask=msk8)
        plsc.store_compressed(idx_v.at[pl.ds(r * _TOP_K, 16)], idx, mask=msk8)
        return carry

    lax.fori_loop(0, _ROWS_PER_W, row_body, 0)

    pltpu.sync_copy(wts_v.at[pl.ds(0, _OUT_W)], wts_hbm.at[pl.ds(wid * _OUT_W, _OUT_W)])
    pltpu.sync_copy(idx_v.at[pl.ds(0, _OUT_W)], idx_hbm.at[pl.ds(wid * _OUT_W, _OUT_W)])


_sc_topk = functools.partial(
    pl.kernel,
    out_type=[
        jax.ShapeDtypeStruct((_N_TOKENS * _TOP_K,), jnp.float32),
        jax.ShapeDtypeStruct((_N_TOKENS * _TOP_K,), jnp.int32),
    ],
    mesh=plsc.VectorSubcoreMesh(core_axis_name="c", subcore_axis_name="s"),
    scratch_types=[
        pltpu.VMEM((_LOG_W,), jnp.float32),
        pltpu.VMEM((_OUT_W + 16,), jnp.float32),
        pltpu.VMEM((_OUT_W + 16,), jnp.int32),
    ],
    compiler_params=pltpu.CompilerParams(needs_layout_passes=False),
)(_sc_kernel)


def kernel(x, W):
    logits = _tc_logits(x, W)
    wts_flat, idx_flat = _sc_topk(logits.reshape(-1))
    return (
        wts_flat.reshape(_N_TOKENS, _TOP_K),
        idx_flat.reshape(_N_TOKENS, _TOP_K),
        logits,
    )


# speedup vs baseline: 1.0335x; 1.0335x over previous
import functools

import jax
import jax.numpy as jnp
from jax import lax
from jax.experimental import pallas as pl
from jax.experimental.pallas import tpu as pltpu
from jax.experimental.pallas import tpu_sc as plsc

_N_TOKENS = 32768
_HIDDEN = 4096
_NUM_EXPERTS = 64
_TOP_K = 8
_BR = 1024

_NW = 32
_NCHUNK = 8
_CHUNK = _N_TOKENS // _NCHUNK
_ROWS_PER_W = _CHUNK // _NW
_LOG_W = _ROWS_PER_W * _NUM_EXPERTS
_OUT_W = _ROWS_PER_W * _TOP_K


def _matmul_kernel(x_ref, w_ref, logits_ref):
    logits_ref[...] = jax.lax.dot_general(
        x_ref[...], w_ref[...], (((1,), (1,)), ((), ())),
        preferred_element_type=jnp.float32,
    )


def _tc_logits_chunk(x, W, c):
    blocks = _CHUNK // _BR
    return pl.pallas_call(
        _matmul_kernel,
        grid=(blocks,),
        in_specs=[
            pl.BlockSpec((_BR, _HIDDEN), lambda i, c=c: (c * blocks + i, 0)),
            pl.BlockSpec((_NUM_EXPERTS, _HIDDEN), lambda i: (0, 0)),
        ],
        out_specs=pl.BlockSpec((_BR, _NUM_EXPERTS), lambda i: (i, 0)),
        out_shape=jax.ShapeDtypeStruct((_CHUNK, _NUM_EXPERTS), jnp.float32),
        compiler_params=pltpu.CompilerParams(
            dimension_semantics=("parallel",),
        ),
    )(x, W)


def _merge16(ak, av, bk, bv):
    rbk = lax.rev(bk, (0,))
    rbv = lax.rev(bv, (0,))
    m = ak >= rbk
    hk = jnp.where(m, ak, rbk)
    hv = jnp.where(m, av, rbv)
    return plsc.sort_key_val(hk, hv, descending=True)


def _sc_kernel(logits_hbm, wts_hbm, idx_hbm, log_v, wts_v, idx_v):
    wid = lax.axis_index("s") * 2 + lax.axis_index("c")
    pltpu.sync_copy(logits_hbm.at[pl.ds(wid * _LOG_W, _LOG_W)], log_v)

    lane = lax.broadcasted_iota(jnp.int32, (16,), 0)
    msk8 = lane < _TOP_K

    def row_body(r, carry):
        off = r * _NUM_EXPERTS
        ks, vs = [], []
        for j in range(4):
            v = log_v[pl.ds(off + 16 * j, 16)]
            b = lax.bitcast_convert_type(v, jnp.int32)
            mono = b ^ ((b >> 31) & jnp.int32(0x7FFFFFFF))
            key = (mono & jnp.int32(~63)) | (jnp.int32(63 - 16 * j) - lane)
            sk, sv = plsc.sort_key_val(key, v, descending=True)
            ks.append(sk)
            vs.append(sv)
        k01, v01 = _merge16(ks[0], vs[0], ks[1], vs[1])
        k23, v23 = _merge16(ks[2], vs[2], ks[3], vs[3])
        kf, vf = _merge16(k01, v01, k23, v23)

        idx = jnp.int32(63) - (kf & jnp.int32(63))
        mx = jnp.max(jnp.where(msk8, vf, -jnp.inf))
        e = jnp.exp(jnp.where(msk8, vf - mx, -jnp.inf))
        w = e / jnp.sum(e)

        plsc.store_compressed(wts_v.at[pl.ds(r * _TOP_K, 16)], w, mask=msk8)
        plsc.store_compressed(idx_v.at[pl.ds(r * _TOP_K, 16)], idx, mask=msk8)
        return carry

    lax.fori_loop(0, _ROWS_PER_W, row_body, 0)

    pltpu.sync_copy(wts_v.at[pl.ds(0, _OUT_W)], wts_hbm.at[pl.ds(wid * _OUT_W, _OUT_W)])
    pltpu.sync_copy(idx_v.at[pl.ds(0, _OUT_W)], idx_hbm.at[pl.ds(wid * _OUT_W, _OUT_W)])


_sc_topk = functools.partial(
    pl.kernel,
    out_type=[
        jax.ShapeDtypeStruct((_CHUNK * _TOP_K,), jnp.float32),
        jax.ShapeDtypeStruct((_CHUNK * _TOP_K,), jnp.int32),
    ],
    mesh=plsc.VectorSubcoreMesh(core_axis_name="c", subcore_axis_name="s"),
    scratch_types=[
        pltpu.VMEM((_LOG_W,), jnp.float32),
        pltpu.VMEM((_OUT_W + 16,), jnp.float32),
        pltpu.VMEM((_OUT_W + 16,), jnp.int32),
    ],
    compiler_params=pltpu.CompilerParams(needs_layout_passes=False),
)(_sc_kernel)


def kernel(x, W):
    logits_c, wts_c, idx_c = [], [], []
    for c in range(_NCHUNK):
        lc = _tc_logits_chunk(x, W, c)
        wf, xf = _sc_topk(lc.reshape(-1))
        logits_c.append(lc)
        wts_c.append(wf.reshape(_CHUNK, _TOP_K))
        idx_c.append(xf.reshape(_CHUNK, _TOP_K))
    return (
        jnp.concatenate(wts_c, axis=0),
        jnp.concatenate(idx_c, axis=0),
        jnp.concatenate(logits_c, axis=0),
    )


# speedup vs baseline: 1.0347x; 1.0011x over previous
import functools

import jax
import jax.numpy as jnp
from jax import lax
from jax.experimental import pallas as pl
from jax.experimental.pallas import tpu as pltpu
from jax.experimental.pallas import tpu_sc as plsc

_N_TOKENS = 32768
_HIDDEN = 4096
_NUM_EXPERTS = 64
_TOP_K = 8
_BR = 1024

_NW = 32
_NCHUNK = 8
_CHUNK = _N_TOKENS // _NCHUNK
_ROWS_PER_W = _CHUNK // _NW
_LOG_W = _ROWS_PER_W * _NUM_EXPERTS
_OUT_W = _ROWS_PER_W * _TOP_K


def _matmul_kernel(x_ref, w_ref, logits_ref):
    logits_ref[...] = jax.lax.dot_general(
        x_ref[...], w_ref[...], (((1,), (1,)), ((), ())),
        preferred_element_type=jnp.float32,
    )


def _tc_logits_chunk(x, W, c):
    blocks = _CHUNK // _BR
    return pl.pallas_call(
        _matmul_kernel,
        grid=(blocks,),
        in_specs=[
            pl.BlockSpec((_BR, _HIDDEN), lambda i, c=c: (c * blocks + i, 0)),
            pl.BlockSpec((_NUM_EXPERTS, _HIDDEN), lambda i: (0, 0)),
        ],
        out_specs=pl.BlockSpec((_BR, _NUM_EXPERTS), lambda i: (i, 0)),
        out_shape=jax.ShapeDtypeStruct((_CHUNK, _NUM_EXPERTS), jnp.float32),
        compiler_params=pltpu.CompilerParams(
            dimension_semantics=("parallel",),
        ),
    )(x, W)


def _merge16(ak, av, bk, bv):
    rbk = lax.rev(bk, (0,))
    rbv = lax.rev(bv, (0,))
    m = ak >= rbk
    hk = jnp.where(m, ak, rbk)
    hv = jnp.where(m, av, rbv)
    return plsc.sort_key_val(hk, hv, descending=True)


def _sc_kernel(logits_hbm, wts_hbm, idx_hbm, log_v, wts_v, idx_v):
    wid = lax.axis_index("s") * 2 + lax.axis_index("c")
    pltpu.sync_copy(logits_hbm.at[pl.ds(wid * _LOG_W, _LOG_W)], log_v)

    lane = lax.broadcasted_iota(jnp.int32, (16,), 0)
    msk8 = lane < _TOP_K

    @plsc.parallel_loop(0, _ROWS_PER_W, unroll=4)
    def row_body(r):
        off = r * _NUM_EXPERTS
        ks, vs = [], []
        for j in range(4):
            v = log_v[pl.ds(off + 16 * j, 16)]
            b = lax.bitcast_convert_type(v, jnp.int32)
            mono = b ^ ((b >> 31) & jnp.int32(0x7FFFFFFF))
            key = (mono & jnp.int32(~63)) | (jnp.int32(63 - 16 * j) - lane)
            sk, sv = plsc.sort_key_val(key, v, descending=True)
            ks.append(sk)
            vs.append(sv)
        k01, v01 = _merge16(ks[0], vs[0], ks[1], vs[1])
        k23, v23 = _merge16(ks[2], vs[2], ks[3], vs[3])
        kf, vf = _merge16(k01, v01, k23, v23)

        idx = jnp.int32(63) - (kf & jnp.int32(63))
        mx = jnp.max(jnp.where(msk8, vf, -jnp.inf))
        e = jnp.exp(jnp.where(msk8, vf - mx, -jnp.inf))
        w = e / jnp.sum(e)

        plsc.store_compressed(wts_v.at[pl.ds(r * _TOP_K, 16)], w, mask=msk8)
        plsc.store_compressed(idx_v.at[pl.ds(r * _TOP_K, 16)], idx, mask=msk8)

    pltpu.sync_copy(wts_v.at[pl.ds(0, _OUT_W)], wts_hbm.at[pl.ds(wid * _OUT_W, _OUT_W)])
    pltpu.sync_copy(idx_v.at[pl.ds(0, _OUT_W)], idx_hbm.at[pl.ds(wid * _OUT_W, _OUT_W)])


_sc_topk = functools.partial(
    pl.kernel,
    out_type=[
        jax.ShapeDtypeStruct((_CHUNK * _TOP_K,), jnp.float32),
        jax.ShapeDtypeStruct((_CHUNK * _TOP_K,), jnp.int32),
    ],
    mesh=plsc.VectorSubcoreMesh(core_axis_name="c", subcore_axis_name="s"),
    scratch_types=[
        pltpu.VMEM((_LOG_W,), jnp.float32),
        pltpu.VMEM((_OUT_W + 16,), jnp.float32),
        pltpu.VMEM((_OUT_W + 16,), jnp.int32),
    ],
    compiler_params=pltpu.CompilerParams(needs_layout_passes=False),
)(_sc_kernel)


def kernel(x, W):
    logits_c, wts_c, idx_c = [], [], []
    for c in range(_NCHUNK):
        lc = _tc_logits_chunk(x, W, c)
        wf, xf = _sc_topk(lc.reshape(-1))
        logits_c.append(lc)
        wts_c.append(wf.reshape(_CHUNK, _TOP_K))
        idx_c.append(xf.reshape(_CHUNK, _TOP_K))
    return (
        jnp.concatenate(wts_c, axis=0),
        jnp.concatenate(idx_c, axis=0),
        jnp.concatenate(logits_c, axis=0),
    )


# speedup vs baseline: 1.5217x; 1.4707x over previous
import jax
import jax.numpy as jnp
from jax.experimental import pallas as pl
from jax.experimental.pallas import tpu as pltpu

_N_TOKENS = 32768
_HIDDEN = 4096
_NUM_EXPERTS = 64
_TOP_K = 8
_BR = 1024


def _gating_kernel(x_ref, w_ref, logits_ref, wts_ref, idx_ref):
    x = x_ref[...]
    w = w_ref[...]
    lt = jax.lax.dot_general(
        w, x, (((1,), (1,)), ((), ())), preferred_element_type=jnp.float32
    )
    logits_ref[...] = lt.T

    erow = jax.lax.broadcasted_iota(jnp.int32, (_NUM_EXPERTS, _BR), 0)
    bits = jax.lax.bitcast_convert_type(lt, jnp.int32)
    mono = bits ^ ((bits >> 31) & jnp.int32(0x7FFFFFFF))
    key = (mono & jnp.int32(~63)) | (jnp.int32(63) - erow)

    neg_inf = jnp.float32(-jnp.inf)
    vals, idxs = [], []
    for _ in range(_TOP_K):
        kmax = jnp.max(key, axis=0, keepdims=True)
        m = key == kmax
        idxs.append(jnp.int32(63) - (kmax & jnp.int32(63)))
        vals.append(jnp.max(jnp.where(m, lt, neg_inf), axis=0, keepdims=True))
        key = jnp.where(m, jnp.int32(-(2**31)), key)

    for k in range(_TOP_K - 1):
        a, b = vals[k], vals[k + 1]
        ia, ib = idxs[k], idxs[k + 1]
        sw = a < b
        vals[k] = jnp.where(sw, b, a)
        vals[k + 1] = jnp.where(sw, a, b)
        idxs[k] = jnp.where(sw, ib, ia)
        idxs[k + 1] = jnp.where(sw, ia, ib)

    topv = jnp.concatenate(vals, axis=0)
    topi = jnp.concatenate(idxs, axis=0)
    e = jnp.exp(topv - topv[:1])
    wts_ref[...] = (e / jnp.sum(e, axis=0, keepdims=True)).T
    idx_ref[...] = topi.T


def kernel(x, W):
    grid = (_N_TOKENS // _BR,)
    logits, wts, idx = pl.pallas_call(
        _gating_kernel,
        grid=grid,
        in_specs=[
            pl.BlockSpec((_BR, _HIDDEN), lambda i: (i, 0)),
            pl.BlockSpec((_NUM_EXPERTS, _HIDDEN), lambda i: (0, 0)),
        ],
        out_specs=[
            pl.BlockSpec((_BR, _NUM_EXPERTS), lambda i: (i, 0)),
            pl.BlockSpec((_BR, _TOP_K), lambda i: (i, 0)),
            pl.BlockSpec((_BR, _TOP_K), lambda i: (i, 0)),
        ],
        out_shape=[
            jax.ShapeDtypeStruct((_N_TOKENS, _NUM_EXPERTS), jnp.float32),
            jax.ShapeDtypeStruct((_N_TOKENS, _TOP_K), jnp.float32),
            jax.ShapeDtypeStruct((_N_TOKENS, _TOP_K), jnp.int32),
        ],
        compiler_params=pltpu.CompilerParams(
            dimension_semantics=("parallel",),
        ),
    )(x, W)
    return (wts, idx, logits)


# speedup vs baseline: 1.5225x; 1.0005x over previous
import jax
import jax.numpy as jnp
from jax.experimental import pallas as pl
from jax.experimental.pallas import tpu as pltpu

_N_TOKENS = 32768
_HIDDEN = 4096
_NUM_EXPERTS = 64
_TOP_K = 8
_BR = 1024


def _gating_kernel(x_ref, w_ref, logits_ref, wts_ref, idx_ref):
    x = x_ref[...]
    w = w_ref[...]
    logits = jax.lax.dot_general(
        x, w, (((1,), (1,)), ((), ())), preferred_element_type=jnp.float32
    )
    logits_ref[...] = logits
    lt = logits.T

    erow = jax.lax.broadcasted_iota(jnp.int32, (_NUM_EXPERTS, _BR), 0)
    bits = jax.lax.bitcast_convert_type(lt, jnp.int32)
    mono = bits ^ ((bits >> 31) & jnp.int32(0x7FFFFFFF))
    key = (mono & jnp.int32(~63)) | (jnp.int32(63) - erow)

    neg_inf = jnp.float32(-jnp.inf)
    vals, idxs = [], []
    for _ in range(_TOP_K):
        kmax = jnp.max(key, axis=0, keepdims=True)
        m = key == kmax
        idxs.append(jnp.int32(63) - (kmax & jnp.int32(63)))
        vals.append(jnp.max(jnp.where(m, lt, neg_inf), axis=0, keepdims=True))
        key = jnp.where(m, jnp.int32(-(2**31)), key)

    for k in range(_TOP_K - 1):
        a, b = vals[k], vals[k + 1]
        ia, ib = idxs[k], idxs[k + 1]
        sw = a < b
        vals[k] = jnp.where(sw, b, a)
        vals[k + 1] = jnp.where(sw, a, b)
        idxs[k] = jnp.where(sw, ib, ia)
        idxs[k + 1] = jnp.where(sw, ia, ib)

    topv = jnp.concatenate(vals, axis=0)
    topi = jnp.concatenate(idxs, axis=0)
    e = jnp.exp(topv - topv[:1])
    wts_ref[...] = (e / jnp.sum(e, axis=0, keepdims=True)).T
    idx_ref[...] = topi.T


def kernel(x, W):
    grid = (_N_TOKENS // _BR,)
    logits, wts, idx = pl.pallas_call(
        _gating_kernel,
        grid=grid,
        in_specs=[
            pl.BlockSpec((_BR, _HIDDEN), lambda i: (i, 0)),
            pl.BlockSpec((_NUM_EXPERTS, _HIDDEN), lambda i: (0, 0)),
        ],
        out_specs=[
            pl.BlockSpec((_BR, _NUM_EXPERTS), lambda i: (i, 0)),
            pl.BlockSpec((_BR, _TOP_K), lambda i: (i, 0)),
            pl.BlockSpec((_BR, _TOP_K), lambda i: (i, 0)),
        ],
        out_shape=[
            jax.ShapeDtypeStruct((_N_TOKENS, _NUM_EXPERTS), jnp.float32),
            jax.ShapeDtypeStruct((_N_TOKENS, _TOP_K), jnp.float32),
            jax.ShapeDtypeStruct((_N_TOKENS, _TOP_K), jnp.int32),
        ],
        compiler_params=pltpu.CompilerParams(
            dimension_semantics=("parallel",),
        ),
    )(x, W)
    return (wts, idx, logits)
